# baseline (speedup 1.0000x reference)
import functools

import jax
import jax.numpy as jnp
from jax.experimental import pallas as pl
from jax.experimental.pallas import tpu as pltpu


def _vb_kernel(Hf, Wf, bbox_ref, obj_ref, frame_ref, wobj_hbm, bobj_ref,
               wkey_hbm, bkey_ref, wval_hbm, bval_ref, wctx_hbm, bctx_ref,
               out_ref, att_acc_ref, wkey_v, wval_v, wobj_v, wctx_v, sems):
    b = pl.program_id(0)
    nb = pl.num_programs(0)
    f32 = jnp.float32
    bf16 = jnp.bfloat16
    N = bbox_ref.shape[0]
    C, P = frame_ref.shape[1], frame_ref.shape[2]

    cp_obj = pltpu.make_async_copy(wobj_hbm, wobj_v, sems.at[0])
    cp_key = pltpu.make_async_copy(wkey_hbm, wkey_v, sems.at[1])
    cp_val = pltpu.make_async_copy(wval_hbm, wval_v, sems.at[2])
    cp_ctx = pltpu.make_async_copy(wctx_hbm, wctx_v, sems.at[3])

    @pl.when(b == 0)
    def _():
        cp_obj.start()
        cp_key.start()
        cp_val.start()
        cp_ctx.start()

    bx = bbox_ref[...]
    x1 = jnp.minimum(bx[:, 0:1], bx[:, 2:3])
    x2 = jnp.maximum(bx[:, 0:1], bx[:, 2:3])
    y1 = jnp.minimum(bx[:, 1:2], bx[:, 3:4])
    y2 = jnp.maximum(bx[:, 1:2], bx[:, 3:4])
    pos = jax.lax.broadcasted_iota(jnp.int32, (N, P), 1)
    yc = ((pos // Wf).astype(f32) + 0.5) * (1.0 / Hf)
    xc = ((pos % Wf).astype(f32) + 0.5) * (1.0 / Wf)
    mask = ((yc >= y1) & (yc <= y2) & (xc >= x1) & (xc <= x2)).astype(f32)
    denom = jnp.maximum(jnp.sum(mask, axis=1, keepdims=True), 1.0)
    onehot = (obj_ref[...] == b).astype(f32)
    mb = (mask * onehot).astype(bf16)

    frame_b = frame_ref[0].astype(bf16)

    pooled = jax.lax.dot_general(mb, frame_b, (((1,), (1,)), ((), ())),
                                 preferred_element_type=f32) / denom

    @pl.when(b == 0)
    def _():
        out_ref[:, :C] = pooled
        cp_obj.wait()
        cp_key.wait()
        cp_val.wait()

    @pl.when(b != 0)
    def _():
        out_ref[:, :C] += pooled

    q = jnp.maximum(
        jnp.dot(pooled.astype(bf16), wobj_v[...].astype(bf16),
                preferred_element_type=f32) + bobj_ref[...], 0.0)
    keym = jnp.maximum(
        jax.lax.dot_general(frame_b, wkey_v[...].astype(bf16),
                            (((0,), (0,)), ((), ())),
                            preferred_element_type=f32) + bkey_ref[...], 0.0)
    valm = jnp.maximum(
        jax.lax.dot_general(frame_b, wval_v[...].astype(bf16),
                            (((0,), (0,)), ((), ())),
                            preferred_element_type=f32) + bval_ref[...], 0.0)
    scores = jax.lax.dot_general(q, keym, (((1,), (1,)), ((), ())),
                                 preferred_element_type=f32)
    m = jnp.max(scores, axis=1, keepdims=True)
    e = jnp.exp(scores - m)
    attn = (e / jnp.sum(e, axis=1, keepdims=True)) * onehot
    att = jnp.dot(attn, valm, preferred_element_type=f32)

    @pl.when(b == 0)
    def _():
        att_acc_ref[...] = att

    @pl.when(b != 0)
    def _():
        att_acc_ref[...] += att

    @pl.when(b == nb - 1)
    def _():
        cp_ctx.wait()
        ctx = jnp.maximum(
            jnp.dot(att_acc_ref[...].astype(bf16), wctx_v[...].astype(bf16),
                    preferred_element_type=f32) + bctx_ref[...], 0.0)
        out_ref[:, C:] = ctx


@jax.jit
def kernel(frame_deep_features, bboxes, obj_slicing, W_obj, b_obj, W_key,
           b_key, W_val, b_val, W_ctx, b_ctx):
    B, C, Hf, Wf = frame_deep_features.shape
    N = bboxes.shape[0]
    P = Hf * Wf
    Dq = W_obj.shape[1]
    Dc = W_ctx.shape[1]
    frame_flat = frame_deep_features.reshape(B, C, P)
    obj2 = obj_slicing.reshape(N, 1)
    anyspec = pl.BlockSpec(memory_space=pl.ANY)

    return pl.pallas_call(
        functools.partial(_vb_kernel, Hf, Wf),
        grid=(B,),
        in_specs=[
            pl.BlockSpec((N, 4), lambda b: (0, 0)),
            pl.BlockSpec((N, 1), lambda b: (0, 0)),
            pl.BlockSpec((1, C, P), lambda b: (b, 0, 0)),
            anyspec,
            pl.BlockSpec((1, Dq), lambda b: (0, 0)),
            anyspec,
            pl.BlockSpec((1, Dq), lambda b: (0, 0)),
            anyspec,
            pl.BlockSpec((1, Dq), lambda b: (0, 0)),
            anyspec,
            pl.BlockSpec((1, Dc), lambda b: (0, 0)),
        ],
        out_specs=pl.BlockSpec((N, C + Dc), lambda b: (0, 0)),
        out_shape=jax.ShapeDtypeStruct((N, C + Dc), jnp.float32),
        scratch_shapes=[
            pltpu.VMEM((N, Dq), jnp.float32),
            pltpu.VMEM((C, Dq), jnp.float32),
            pltpu.VMEM((C, Dq), jnp.float32),
            pltpu.VMEM((C, Dq), jnp.float32),
            pltpu.VMEM((Dq, Dc), jnp.float32),
            pltpu.SemaphoreType.DMA((4,)),
        ],
    )(bboxes, obj2, frame_flat, W_obj, b_obj.reshape(1, Dq),
      W_key, b_key.reshape(1, Dq), W_val,
      b_val.reshape(1, Dq), W_ctx, b_ctx.reshape(1, Dc))


# speedup vs baseline: 1.0847x; 1.0847x over previous
import functools

import jax
import jax.numpy as jnp
from jax.experimental import pallas as pl
from jax.experimental.pallas import tpu as pltpu


def _vb_kernel(Hf, Wf, bbox_ref, obj_ref, frame_ref, wobj_hbm, bobj_ref,
               wkey_hbm, bkey_ref, wval_hbm, bval_ref, wctx_hbm, bctx_ref,
               out_ref, wkey_v, wval_v, wobj_v, wctx_v, key_ref, val_ref,
               sems):
    b = pl.program_id(0)
    nb = pl.num_programs(0)
    f32 = jnp.float32
    bf16 = jnp.bfloat16
    N = bbox_ref.shape[0]
    C, P = frame_ref.shape[1], frame_ref.shape[2]
    Dq = wobj_v.shape[1]

    cp_key = pltpu.make_async_copy(wkey_hbm, wkey_v, sems.at[0])
    cp_val = pltpu.make_async_copy(wval_hbm, wval_v, sems.at[1])
    cp_obj = pltpu.make_async_copy(wobj_hbm, wobj_v, sems.at[2])
    cp_ctx = pltpu.make_async_copy(wctx_hbm, wctx_v, sems.at[3])

    @pl.when(b == 0)
    def _():
        cp_key.start()
        cp_val.start()
        cp_obj.start()
        cp_ctx.start()

    bx = bbox_ref[...]
    x1 = jnp.minimum(bx[:, 0:1], bx[:, 2:3])
    x2 = jnp.maximum(bx[:, 0:1], bx[:, 2:3])
    y1 = jnp.minimum(bx[:, 1:2], bx[:, 3:4])
    y2 = jnp.maximum(bx[:, 1:2], bx[:, 3:4])
    pos = jax.lax.broadcasted_iota(jnp.int32, (N, P), 1)
    yc = ((pos // Wf).astype(f32) + 0.5) * (1.0 / Hf)
    xc = ((pos % Wf).astype(f32) + 0.5) * (1.0 / Wf)
    mask = ((yc >= y1) & (yc <= y2) & (xc >= x1) & (xc <= x2)).astype(f32)
    denom = jnp.maximum(jnp.sum(mask, axis=1, keepdims=True), 1.0)
    onehot = (obj_ref[...] == b).astype(f32)
    mb = (mask * onehot).astype(bf16)

    frame_b = frame_ref[0].astype(bf16)

    pooled = jax.lax.dot_general(mb, frame_b, (((1,), (1,)), ((), ())),
                                 preferred_element_type=f32) / denom

    @pl.when(b == 0)
    def _():
        out_ref[:, :C] = pooled
        cp_key.wait()

    @pl.when(b != 0)
    def _():
        out_ref[:, :C] += pooled

    keym = jnp.maximum(
        jax.lax.dot_general(frame_b, wkey_v[...].astype(bf16),
                            (((0,), (0,)), ((), ())),
                            preferred_element_type=f32) + bkey_ref[...], 0.0)
    key_ref[pl.ds(b * P, P), :] = keym.astype(bf16)

    @pl.when(b == 0)
    def _():
        cp_val.wait()

    valm = jnp.maximum(
        jax.lax.dot_general(frame_b, wval_v[...].astype(bf16),
                            (((0,), (0,)), ((), ())),
                            preferred_element_type=f32) + bval_ref[...], 0.0)
    val_ref[pl.ds(b * P, P), :] = valm.astype(bf16)

    @pl.when(b == nb - 1)
    def _():
        cp_obj.wait()
        q = jnp.maximum(
            jnp.dot(out_ref[:, :C].astype(bf16), wobj_v[...].astype(bf16),
                    preferred_element_type=f32) + bobj_ref[...], 0.0)
        scores = jax.lax.dot_general(
            q.astype(bf16), key_ref[...], (((1,), (1,)), ((), ())),
            preferred_element_type=f32)
        seg = jax.lax.broadcasted_iota(jnp.int32, (N, nb * P), 1) // P
        scores = jnp.where(seg == obj_ref[...], scores, -jnp.inf)
        m = jnp.max(scores, axis=1, keepdims=True)
        e = jnp.exp(scores - m)
        attn = e / jnp.sum(e, axis=1, keepdims=True)
        att = jnp.dot(attn.astype(bf16), val_ref[...],
                      preferred_element_type=f32)
        cp_ctx.wait()
        ctx = jnp.maximum(
            jnp.dot(att.astype(bf16), wctx_v[...].astype(bf16),
                    preferred_element_type=f32) + bctx_ref[...], 0.0)
        out_ref[:, C:] = ctx


@jax.jit
def kernel(frame_deep_features, bboxes, obj_slicing, W_obj, b_obj, W_key,
           b_key, W_val, b_val, W_ctx, b_ctx):
    B, C, Hf, Wf = frame_deep_features.shape
    N = bboxes.shape[0]
    P = Hf * Wf
    Dq = W_obj.shape[1]
    Dc = W_ctx.shape[1]
    frame_flat = frame_deep_features.reshape(B, C, P)
    obj2 = obj_slicing.reshape(N, 1)
    anyspec = pl.BlockSpec(memory_space=pl.ANY)

    return pl.pallas_call(
        functools.partial(_vb_kernel, Hf, Wf),
        grid=(B,),
        in_specs=[
            pl.BlockSpec((N, 4), lambda b: (0, 0)),
            pl.BlockSpec((N, 1), lambda b: (0, 0)),
            pl.BlockSpec((1, C, P), lambda b: (b, 0, 0)),
            anyspec,
            pl.BlockSpec((1, Dq), lambda b: (0, 0)),
            anyspec,
            pl.BlockSpec((1, Dq), lambda b: (0, 0)),
            anyspec,
            pl.BlockSpec((1, Dq), lambda b: (0, 0)),
            anyspec,
            pl.BlockSpec((1, Dc), lambda b: (0, 0)),
        ],
        out_specs=pl.BlockSpec((N, C + Dc), lambda b: (0, 0)),
        out_shape=jax.ShapeDtypeStruct((N, C + Dc), jnp.float32),
        scratch_shapes=[
            pltpu.VMEM((C, Dq), jnp.float32),
            pltpu.VMEM((C, Dq), jnp.float32),
            pltpu.VMEM((C, Dq), jnp.float32),
            pltpu.VMEM((Dq, Dc), jnp.float32),
            pltpu.VMEM((B * P, Dq), jnp.bfloat16),
            pltpu.VMEM((B * P, Dq), jnp.bfloat16),
            pltpu.SemaphoreType.DMA((4,)),
        ],
    )(bboxes, obj2, frame_flat, W_obj, b_obj.reshape(1, Dq),
      W_key, b_key.reshape(1, Dq), W_val,
      b_val.reshape(1, Dq), W_ctx, b_ctx.reshape(1, Dc))


# speedup vs baseline: 1.1890x; 1.0962x over previous
import functools

import jax
import jax.numpy as jnp
from jax.experimental import pallas as pl
from jax.experimental.pallas import tpu as pltpu


def _vb_kernel(B, Hf, Wf, bbox_ref, obj_ref, frame_hbm, wobj_hbm, bobj_ref,
               wkey_hbm, bkey_ref, wval_hbm, bval_ref, wctx_hbm, bctx_ref,
               out_ref, fb0_ref, fb1_ref, wkey_v, wval_v, wobj_v, wctx_v,
               key_ref, val_ref, sems):
    f32 = jnp.float32
    bf16 = jnp.bfloat16
    N = bbox_ref.shape[0]
    C, P = frame_hbm.shape[1], frame_hbm.shape[2]
    Dq = wobj_v.shape[1]
    fbufs = (fb0_ref, fb1_ref)

    cp_f = [pltpu.make_async_copy(frame_hbm.at[i], fbufs[i % 2].at[0],
                                  sems.at[i]) for i in range(B)]
    cp_key = pltpu.make_async_copy(wkey_hbm, wkey_v, sems.at[4])
    cp_val = pltpu.make_async_copy(wval_hbm, wval_v, sems.at[5])
    cp_obj = pltpu.make_async_copy(wobj_hbm, wobj_v, sems.at[6])
    cp_ctx = pltpu.make_async_copy(wctx_hbm, wctx_v, sems.at[7])

    cp_f[0].start()
    cp_key.start()
    cp_val.start()
    cp_f[1].start()

    bx = bbox_ref[...]
    x1 = jnp.minimum(bx[:, 0:1], bx[:, 2:3])
    x2 = jnp.maximum(bx[:, 0:1], bx[:, 2:3])
    y1 = jnp.minimum(bx[:, 1:2], bx[:, 3:4])
    y2 = jnp.maximum(bx[:, 1:2], bx[:, 3:4])
    pos = jax.lax.broadcasted_iota(jnp.int32, (N, P), 1)
    yc = ((pos // Wf).astype(f32) + 0.5) * (1.0 / Hf)
    xc = ((pos % Wf).astype(f32) + 0.5) * (1.0 / Wf)
    mask = ((yc >= y1) & (yc <= y2) & (xc >= x1) & (xc <= x2)).astype(f32)
    inv_denom = 1.0 / jnp.maximum(jnp.sum(mask, axis=1, keepdims=True), 1.0)

    for b in range(B):
        onehot = (obj_ref[...] == b).astype(f32)
        mb = (mask * onehot).astype(bf16)
        cp_f[b].wait()
        frame_b = fbufs[b % 2][0].astype(bf16)
        pooled = jax.lax.dot_general(
            mb, frame_b, (((1,), (1,)), ((), ())),
            preferred_element_type=f32) * inv_denom
        if b == 0:
            out_ref[:, :C] = pooled
            cp_key.wait()
        else:
            out_ref[:, :C] += pooled
        keym = jnp.maximum(
            jax.lax.dot_general(frame_b, wkey_v[...].astype(bf16),
                                (((0,), (0,)), ((), ())),
                                preferred_element_type=f32)
            + bkey_ref[...], 0.0)
        key_ref[b * P:(b + 1) * P, :] = keym.astype(bf16)
        if b == 0:
            cp_val.wait()
        valm = jnp.maximum(
            jax.lax.dot_general(frame_b, wval_v[...].astype(bf16),
                                (((0,), (0,)), ((), ())),
                                preferred_element_type=f32)
            + bval_ref[...], 0.0)
        val_ref[b * P:(b + 1) * P, :] = valm.astype(bf16)
        if b == 0:
            cp_f[2].start()
        elif b == 1:
            cp_f[3].start()
            cp_obj.start()
        elif b == 2:
            cp_ctx.start()

    cp_obj.wait()
    q = jnp.maximum(
        jnp.dot(out_ref[:, :C].astype(bf16), wobj_v[...].astype(bf16),
                preferred_element_type=f32) + bobj_ref[...], 0.0)
    scores = jax.lax.dot_general(
        q.astype(bf16), key_ref[...], (((1,), (1,)), ((), ())),
        preferred_element_type=f32)
    seg = jax.lax.broadcasted_iota(jnp.int32, (N, B * P), 1) // P
    scores = jnp.where(seg == obj_ref[...], scores, -jnp.inf)
    m = jnp.max(scores, axis=1, keepdims=True)
    e = jnp.exp(scores - m)
    attn = e / jnp.sum(e, axis=1, keepdims=True)
    att = jnp.dot(attn.astype(bf16), val_ref[...],
                  preferred_element_type=f32)
    cp_ctx.wait()
    ctx = jnp.maximum(
        jnp.dot(att.astype(bf16), wctx_v[...].astype(bf16),
                preferred_element_type=f32) + bctx_ref[...], 0.0)
    out_ref[:, C:] = ctx


@jax.jit
def kernel(frame_deep_features, bboxes, obj_slicing, W_obj, b_obj, W_key,
           b_key, W_val, b_val, W_ctx, b_ctx):
    B, C, Hf, Wf = frame_deep_features.shape
    N = bboxes.shape[0]
    P = Hf * Wf
    Dq = W_obj.shape[1]
    Dc = W_ctx.shape[1]
    frame_flat = frame_deep_features.reshape(B, C, P)
    obj2 = obj_slicing.reshape(N, 1)
    anyspec = pl.BlockSpec(memory_space=pl.ANY)

    return pl.pallas_call(
        functools.partial(_vb_kernel, B, Hf, Wf),
        in_specs=[
            pl.BlockSpec((N, 4), lambda: (0, 0)),
            pl.BlockSpec((N, 1), lambda: (0, 0)),
            anyspec,
            anyspec,
            pl.BlockSpec((1, Dq), lambda: (0, 0)),
            anyspec,
            pl.BlockSpec((1, Dq), lambda: (0, 0)),
            anyspec,
            pl.BlockSpec((1, Dq), lambda: (0, 0)),
            anyspec,
            pl.BlockSpec((1, Dc), lambda: (0, 0)),
        ],
        out_specs=pl.BlockSpec((N, C + Dc), lambda: (0, 0)),
        out_shape=jax.ShapeDtypeStruct((N, C + Dc), jnp.float32),
        scratch_shapes=[
            pltpu.VMEM((1, C, P), jnp.float32),
            pltpu.VMEM((1, C, P), jnp.float32),
            pltpu.VMEM((C, Dq), jnp.float32),
            pltpu.VMEM((C, Dq), jnp.float32),
            pltpu.VMEM((C, Dq), jnp.float32),
            pltpu.VMEM((Dq, Dc), jnp.float32),
            pltpu.VMEM((B * P, Dq), jnp.bfloat16),
            pltpu.VMEM((B * P, Dq), jnp.bfloat16),
            pltpu.SemaphoreType.DMA((8,)),
        ],
    )(bboxes, obj2, frame_flat, W_obj, b_obj.reshape(1, Dq),
      W_key, b_key.reshape(1, Dq), W_val,
      b_val.reshape(1, Dq), W_ctx, b_ctx.reshape(1, Dc))
